# baseline (speedup 1.0000x reference)
import functools

import jax
import jax.numpy as jnp
import numpy as np
from jax import lax
from jax.experimental import pallas as pl
from jax.experimental.pallas import tpu as pltpu
from jax.experimental.pallas import tpu_sc as plsc

_N = 32768
_NCLASS = 128
_B = 128
_T = 50
_EPS = 1e-30

_betas = np.array([0.0004 * (k + 1) for k in range(_T)], dtype=np.float64)
_alphas = 1.0 - _betas
_alpha_bars = np.cumprod(_alphas, axis=0)
_prev_alpha_bars = np.concatenate([[1.0], _alpha_bars[:-1]])
_LA = np.log(_alphas + _EPS).astype(np.float32)
_L1MA = np.log(1.0 - _alphas + _EPS).astype(np.float32)
_LPAB = np.log(_prev_alpha_bars + _EPS).astype(np.float32)
_L1MPAB = np.log(1.0 - _prev_alpha_bars + _EPS).astype(np.float32)
_C_INIT = np.float32(np.maximum(np.log(1.0 / _NCLASS + _EPS), -30.0))
_LOG_EPS30 = np.log(np.float32(_EPS))


_TABLE = np.zeros((64, 8), dtype=np.float32)
_TABLE[:_T, 0] = _LA
_TABLE[:_T, 1] = (_LOG_EPS30 + _LA).astype(np.float32)
_TABLE[:_T, 2] = (_L1MA + _C_INIT).astype(np.float32)
_TABLE[:_T, 3] = _LPAB
_TABLE[:_T, 4] = (_L1MPAB + _C_INIT).astype(np.float32)
_TABLE[0, 5] = 1.0


_NW = 32
_BPW = _N // _NW


@functools.cache
def _make_sc_gather_t():
    @functools.partial(
        pl.kernel,
        mesh=plsc.VectorSubcoreMesh(core_axis_name="c", subcore_axis_name="s"),
        out_type=jax.ShapeDtypeStruct((_N,), jnp.int32),
        scratch_types=[
            pltpu.VMEM((_B,), jnp.int32),
            pltpu.VMEM((_BPW,), jnp.int32),
            pltpu.VMEM((_BPW,), jnp.int32),
        ],
        compiler_params=pltpu.CompilerParams(needs_layout_passes=False),
    )
    def _sc_gather_t(ts_hbm, batch_hbm, out_hbm, ts_v, b_v, t_v):
        wid = lax.axis_index("s") * 2 + lax.axis_index("c")
        base = wid * _BPW
        pltpu.sync_copy(ts_hbm, ts_v)
        pltpu.sync_copy(batch_hbm.at[pl.ds(base, _BPW)], b_v)
        for g in range(_BPW // 16):
            idx = b_v[pl.ds(g * 16, 16)]
            t_v[pl.ds(g * 16, 16)] = plsc.load_gather(ts_v, [idx])
        pltpu.sync_copy(t_v, out_hbm.at[pl.ds(base, _BPW)])

    return _sc_gather_t


_CH = 256


def _body(x0_ref, u_ref, xt_ref, t_ref, tab_ref, lp_ref, s_ref):
    r = x0_ref.shape[0]
    lane = jax.lax.broadcasted_iota(jnp.int32, (1, _NCLASS), 1)

    tab = tab_ref[...]
    la_c = tab[:, 0:1]
    lae_c = tab[:, 1:2]
    l1_c = tab[:, 2:3]
    m1 = jnp.maximum(la_c, l1_c)
    lxt_c = m1 + jnp.log(jnp.exp(la_c - m1) + jnp.exp(l1_c - m1))
    m2 = jnp.maximum(lae_c, l1_c)
    loth_c = m2 + jnp.log(jnp.exp(lae_c - m2) + jnp.exp(l1_c - m2))
    mat = jnp.concatenate([lxt_c, loth_c, tab[:, 3:6]], axis=1)
    lane64 = jax.lax.broadcasted_iota(jnp.int32, (1, 64), 1)

    def _chunk(i, carry):
        sl = pl.ds(i * _CH, _CH)
        x0 = x0_ref[sl, :]
        oh_t = (t_ref[sl, :] == lane64).astype(jnp.float32)
        coefs = jax.lax.dot_general(oh_t, mat, (((1,), (0,)), ((), ())),
                                    precision=jax.lax.Precision.HIGHEST,
                                    preferred_element_type=jnp.float32)
        lxt = coefs[:, 0:1]
        loth = coefs[:, 1:2]
        lpab = coefs[:, 2:3]
        r2 = coefs[:, 3:4]
        tzm = coefs[:, 4:5] > 0.5

        xtm = xt_ref[sl, :] == lane
        left = jnp.where(xtm, lxt, loth)
        a = lpab + x0
        m = jnp.maximum(a, r2)
        right = m + jnp.log(jnp.exp(a - m) + jnp.exp(r2 - m))
        lp = left + right
        rowmax = jnp.max(lp, axis=1, keepdims=True)
        lse = rowmax + jnp.log(
            jnp.sum(jnp.exp(lp - rowmax), axis=1, keepdims=True))
        lp_out = jnp.where(tzm, x0, lp - lse)
        lp_ref[sl, :] = lp_out

        g = -jnp.log(-jnp.log(u_ref[sl, :] + jnp.float32(_EPS))
                     + jnp.float32(_EPS))
        sel = jnp.where(tzm, x0, g + lp_out)
        lane_i = jax.lax.broadcasted_iota(jnp.int32, (_CH, _NCLASS), 1)
        smax = jnp.max(sel, axis=1, keepdims=True)
        s_ref[sl, :] = jnp.min(jnp.where(sel == smax, lane_i, _NCLASS),
                               axis=1, keepdims=True)
        return carry

    jax.lax.fori_loop(0, r // _CH, _chunk, 0)


def kernel(x0_logprob, uniform_noise, xt, timestep, batch):
    r = 4096
    grid = (_N // r,)
    tvals = _make_sc_gather_t()(timestep, batch)
    xt2 = xt.reshape(_N, 1)
    t2 = tvals.reshape(_N, 1)
    tab = jnp.asarray(_TABLE)
    lp, s2 = pl.pallas_call(
        _body,
        grid=grid,
        in_specs=[
            pl.BlockSpec((r, _NCLASS), lambda i: (i, 0)),
            pl.BlockSpec((r, _NCLASS), lambda i: (i, 0)),
            pl.BlockSpec((r, 1), lambda i: (i, 0)),
            pl.BlockSpec((r, 1), lambda i: (i, 0)),
            pl.BlockSpec((64, 8), lambda i: (0, 0)),
        ],
        out_specs=[
            pl.BlockSpec((r, _NCLASS), lambda i: (i, 0)),
            pl.BlockSpec((r, 1), lambda i: (i, 0)),
        ],
        out_shape=[
            jax.ShapeDtypeStruct((_N, _NCLASS), jnp.float32),
            jax.ShapeDtypeStruct((_N, 1), jnp.int32),
        ],
        compiler_params=pltpu.CompilerParams(dimension_semantics=("parallel",)),
    )(x0_logprob, uniform_noise, xt2, t2, tab)
    return lp, s2.reshape(_N)


# speedup vs baseline: 1.0318x; 1.0318x over previous
import functools

import jax
import jax.numpy as jnp
import numpy as np
from jax import lax
from jax.experimental import pallas as pl
from jax.experimental.pallas import tpu as pltpu
from jax.experimental.pallas import tpu_sc as plsc

_N = 32768
_NCLASS = 128
_B = 128
_T = 50
_EPS = 1e-30

_betas = np.array([0.0004 * (k + 1) for k in range(_T)], dtype=np.float64)
_alphas = 1.0 - _betas
_alpha_bars = np.cumprod(_alphas, axis=0)
_prev_alpha_bars = np.concatenate([[1.0], _alpha_bars[:-1]])
_LA = np.log(_alphas + _EPS).astype(np.float32)
_L1MA = np.log(1.0 - _alphas + _EPS).astype(np.float32)
_LPAB = np.log(_prev_alpha_bars + _EPS).astype(np.float32)
_L1MPAB = np.log(1.0 - _prev_alpha_bars + _EPS).astype(np.float32)
_C_INIT = np.float32(np.maximum(np.log(1.0 / _NCLASS + _EPS), -30.0))
_LOG_EPS30 = np.log(np.float32(_EPS))


_TABLE = np.zeros((64, 8), dtype=np.float32)
_TABLE[:_T, 0] = _LA
_TABLE[:_T, 1] = (_LOG_EPS30 + _LA).astype(np.float32)
_TABLE[:_T, 2] = (_L1MA + _C_INIT).astype(np.float32)
_TABLE[:_T, 3] = _LPAB
_TABLE[:_T, 4] = (_L1MPAB + _C_INIT).astype(np.float32)
_TABLE[0, 5] = 1.0


_NW = 32
_BPW = _N // _NW


@functools.cache
def _make_sc_gather_t():
    @functools.partial(
        pl.kernel,
        mesh=plsc.VectorSubcoreMesh(core_axis_name="c", subcore_axis_name="s"),
        out_type=jax.ShapeDtypeStruct((_N,), jnp.int32),
        scratch_types=[
            pltpu.VMEM((_B,), jnp.int32),
            pltpu.VMEM((_BPW,), jnp.int32),
            pltpu.VMEM((_BPW,), jnp.int32),
        ],
        compiler_params=pltpu.CompilerParams(needs_layout_passes=False),
    )
    def _sc_gather_t(ts_hbm, batch_hbm, out_hbm, ts_v, b_v, t_v):
        wid = lax.axis_index("s") * 2 + lax.axis_index("c")
        base = wid * _BPW
        pltpu.sync_copy(ts_hbm, ts_v)
        pltpu.sync_copy(batch_hbm.at[pl.ds(base, _BPW)], b_v)
        for g in range(_BPW // 16):
            idx = b_v[pl.ds(g * 16, 16)]
            t_v[pl.ds(g * 16, 16)] = plsc.load_gather(ts_v, [idx])
        pltpu.sync_copy(t_v, out_hbm.at[pl.ds(base, _BPW)])

    return _sc_gather_t


_CH = 256


def _body(x0_ref, u_ref, xt_ref, t_ref, tab_ref, lp_ref, s_ref):
    r = x0_ref.shape[0]
    lane = jax.lax.broadcasted_iota(jnp.int32, (1, _NCLASS), 1)

    tab = tab_ref[...]
    la_c = tab[:, 0:1]
    lae_c = tab[:, 1:2]
    l1_c = tab[:, 2:3]
    m1 = jnp.maximum(la_c, l1_c)
    lxt_c = m1 + jnp.log(jnp.exp(la_c - m1) + jnp.exp(l1_c - m1))
    m2 = jnp.maximum(lae_c, l1_c)
    loth_c = m2 + jnp.log(jnp.exp(lae_c - m2) + jnp.exp(l1_c - m2))
    mat = jnp.concatenate([lxt_c, loth_c, tab[:, 3:6]], axis=1)
    lane64 = jax.lax.broadcasted_iota(jnp.int32, (1, 64), 1)

    def _chunk(i):
        sl = pl.ds(i * _CH, _CH)
        x0 = x0_ref[sl, :]
        oh_t = (t_ref[sl, :] == lane64).astype(jnp.float32)
        coefs = jax.lax.dot_general(oh_t, mat, (((1,), (0,)), ((), ())),
                                    precision=jax.lax.Precision.HIGHEST,
                                    preferred_element_type=jnp.float32)
        lxt = coefs[:, 0:1]
        loth = coefs[:, 1:2]
        lpab = coefs[:, 2:3]
        r2 = coefs[:, 3:4]
        tzm = coefs[:, 4:5] > 0.5

        xtm = xt_ref[sl, :] == lane
        left = jnp.where(xtm, lxt, loth)
        a = lpab + x0
        m = jnp.maximum(a, r2)
        right = m + jnp.log(jnp.exp(a - m) + jnp.exp(r2 - m))
        lp = left + right
        rowmax = jnp.max(lp, axis=1, keepdims=True)
        lse = rowmax + jnp.log(
            jnp.sum(jnp.exp(lp - rowmax), axis=1, keepdims=True))
        lp_out = jnp.where(tzm, x0, lp - lse)
        lp_ref[sl, :] = lp_out

        g = -jnp.log(-jnp.log(u_ref[sl, :] + jnp.float32(_EPS))
                     + jnp.float32(_EPS))
        sel = jnp.where(tzm, x0, g + lp_out)
        lane_i = jax.lax.broadcasted_iota(jnp.int32, (_CH, _NCLASS), 1)
        smax = jnp.max(sel, axis=1, keepdims=True)
        s_ref[sl, :] = jnp.min(jnp.where(sel == smax, lane_i, _NCLASS),
                               axis=1, keepdims=True)

    for i in range(r // _CH):
        _chunk(i)


def kernel(x0_logprob, uniform_noise, xt, timestep, batch):
    r = 4096
    grid = (_N // r,)
    tvals = _make_sc_gather_t()(timestep, batch)
    xt2 = xt.reshape(_N, 1)
    t2 = tvals.reshape(_N, 1)
    tab = jnp.asarray(_TABLE)
    lp, s2 = pl.pallas_call(
        _body,
        grid=grid,
        in_specs=[
            pl.BlockSpec((r, _NCLASS), lambda i: (i, 0)),
            pl.BlockSpec((r, _NCLASS), lambda i: (i, 0)),
            pl.BlockSpec((r, 1), lambda i: (i, 0)),
            pl.BlockSpec((r, 1), lambda i: (i, 0)),
            pl.BlockSpec((64, 8), lambda i: (0, 0)),
        ],
        out_specs=[
            pl.BlockSpec((r, _NCLASS), lambda i: (i, 0)),
            pl.BlockSpec((r, 1), lambda i: (i, 0)),
        ],
        out_shape=[
            jax.ShapeDtypeStruct((_N, _NCLASS), jnp.float32),
            jax.ShapeDtypeStruct((_N, 1), jnp.int32),
        ],
        compiler_params=pltpu.CompilerParams(dimension_semantics=("parallel",)),
    )(x0_logprob, uniform_noise, xt2, t2, tab)
    return lp, s2.reshape(_N)


# speedup vs baseline: 1.5647x; 1.5165x over previous
import functools

import jax
import jax.numpy as jnp
import numpy as np
from jax import lax
from jax.experimental import pallas as pl
from jax.experimental.pallas import tpu as pltpu
from jax.experimental.pallas import tpu_sc as plsc

_N = 32768
_NCLASS = 128
_B = 128
_T = 50
_EPS = 1e-30

_betas = np.array([0.0004 * (k + 1) for k in range(_T)], dtype=np.float64)
_alphas = 1.0 - _betas
_alpha_bars = np.cumprod(_alphas, axis=0)
_prev_alpha_bars = np.concatenate([[1.0], _alpha_bars[:-1]])
_LA = np.log(_alphas + _EPS).astype(np.float32)
_L1MA = np.log(1.0 - _alphas + _EPS).astype(np.float32)
_LPAB = np.log(_prev_alpha_bars + _EPS).astype(np.float32)
_L1MPAB = np.log(1.0 - _prev_alpha_bars + _EPS).astype(np.float32)
_C_INIT = np.float32(np.maximum(np.log(1.0 / _NCLASS + _EPS), -30.0))
_LOG_EPS30 = np.log(np.float32(_EPS))


_TABLE = np.zeros((64, 8), dtype=np.float32)
_TABLE[:_T, 0] = _LA
_TABLE[:_T, 1] = (_LOG_EPS30 + _LA).astype(np.float32)
_TABLE[:_T, 2] = (_L1MA + _C_INIT).astype(np.float32)
_TABLE[:_T, 3] = _LPAB
_TABLE[:_T, 4] = (_L1MPAB + _C_INIT).astype(np.float32)
_TABLE[0, 5] = 1.0


_NW = 32
_BPW = _N // _NW


@functools.cache
def _make_sc_gather_t():
    @functools.partial(
        pl.kernel,
        mesh=plsc.VectorSubcoreMesh(core_axis_name="c", subcore_axis_name="s"),
        out_type=jax.ShapeDtypeStruct((_N,), jnp.int32),
        scratch_types=[
            pltpu.VMEM((_B,), jnp.int32),
            pltpu.VMEM((_BPW,), jnp.int32),
            pltpu.VMEM((_BPW,), jnp.int32),
        ],
        compiler_params=pltpu.CompilerParams(needs_layout_passes=False),
    )
    def _sc_gather_t(ts_hbm, batch_hbm, out_hbm, ts_v, b_v, t_v):
        wid = lax.axis_index("s") * 2 + lax.axis_index("c")
        base = wid * _BPW
        pltpu.sync_copy(ts_hbm, ts_v)
        pltpu.sync_copy(batch_hbm.at[pl.ds(base, _BPW)], b_v)
        for g in range(_BPW // 16):
            idx = b_v[pl.ds(g * 16, 16)]
            t_v[pl.ds(g * 16, 16)] = plsc.load_gather(ts_v, [idx])
        pltpu.sync_copy(t_v, out_hbm.at[pl.ds(base, _BPW)])

    return _sc_gather_t


def _body(x0_ref, u_ref, xt_ref, t_ref, tab_ref, lp_ref, s_ref):
    r = x0_ref.shape[0]
    lane = jax.lax.broadcasted_iota(jnp.int32, (1, _NCLASS), 1)

    tab = tab_ref[...]
    la_c = tab[:, 0:1]
    lae_c = tab[:, 1:2]
    l1_c = tab[:, 2:3]
    m1 = jnp.maximum(la_c, l1_c)
    lxt_c = m1 + jnp.log(jnp.exp(la_c - m1) + jnp.exp(l1_c - m1))
    m2 = jnp.maximum(lae_c, l1_c)
    loth_c = m2 + jnp.log(jnp.exp(lae_c - m2) + jnp.exp(l1_c - m2))
    mat = jnp.concatenate([lxt_c, loth_c, tab[:, 3:6]], axis=1)
    lane64 = jax.lax.broadcasted_iota(jnp.int32, (1, 64), 1)

    x0 = x0_ref[...]
    oh_t = (t_ref[...] == lane64).astype(jnp.float32)
    coefs = jax.lax.dot_general(oh_t, mat, (((1,), (0,)), ((), ())),
                                precision=jax.lax.Precision.HIGHEST,
                                preferred_element_type=jnp.float32)
    lxt = coefs[:, 0:1]
    loth = coefs[:, 1:2]
    lpab = coefs[:, 2:3]
    r2 = coefs[:, 3:4]
    tzm = coefs[:, 4:5] > 0.5

    xtm = xt_ref[...] == lane
    left = jnp.where(xtm, lxt, loth)
    a = lpab + x0
    m = jnp.maximum(a, r2)
    right = m + jnp.log(jnp.exp(a - m) + jnp.exp(r2 - m))
    lp = left + right
    rowmax = jnp.max(lp, axis=1, keepdims=True)
    lse = rowmax + jnp.log(
        jnp.sum(jnp.exp(lp - rowmax), axis=1, keepdims=True))
    lp_out = jnp.where(tzm, x0, lp - lse)
    lp_ref[...] = lp_out

    g = -jnp.log(-jnp.log(u_ref[...] + jnp.float32(_EPS))
                 + jnp.float32(_EPS))
    sel = jnp.where(tzm, x0, g + lp_out)
    lane_i = jax.lax.broadcasted_iota(jnp.int32, (r, _NCLASS), 1)
    smax = jnp.max(sel, axis=1, keepdims=True)
    s_ref[...] = jnp.min(jnp.where(sel == smax, lane_i, _NCLASS),
                         axis=1, keepdims=True)


def kernel(x0_logprob, uniform_noise, xt, timestep, batch):
    r = 4096
    grid = (_N // r,)
    tvals = _make_sc_gather_t()(timestep, batch)
    xt2 = xt.reshape(_N, 1)
    t2 = tvals.reshape(_N, 1)
    tab = jnp.asarray(_TABLE)
    lp, s2 = pl.pallas_call(
        _body,
        grid=grid,
        in_specs=[
            pl.BlockSpec((r, _NCLASS), lambda i: (i, 0)),
            pl.BlockSpec((r, _NCLASS), lambda i: (i, 0)),
            pl.BlockSpec((r, 1), lambda i: (i, 0)),
            pl.BlockSpec((r, 1), lambda i: (i, 0)),
            pl.BlockSpec((64, 8), lambda i: (0, 0)),
        ],
        out_specs=[
            pl.BlockSpec((r, _NCLASS), lambda i: (i, 0)),
            pl.BlockSpec((r, 1), lambda i: (i, 0)),
        ],
        out_shape=[
            jax.ShapeDtypeStruct((_N, _NCLASS), jnp.float32),
            jax.ShapeDtypeStruct((_N, 1), jnp.int32),
        ],
        compiler_params=pltpu.CompilerParams(dimension_semantics=("parallel",)),
    )(x0_logprob, uniform_noise, xt2, t2, tab)
    return lp, s2.reshape(_N)


# speedup vs baseline: 1.7085x; 1.0919x over previous
import functools

import jax
import jax.numpy as jnp
import numpy as np
from jax import lax
from jax.experimental import pallas as pl
from jax.experimental.pallas import tpu as pltpu
from jax.experimental.pallas import tpu_sc as plsc

_N = 32768
_NCLASS = 128
_B = 128
_T = 50
_EPS = 1e-30

_betas = np.array([0.0004 * (k + 1) for k in range(_T)], dtype=np.float64)
_alphas = 1.0 - _betas
_alpha_bars = np.cumprod(_alphas, axis=0)
_prev_alpha_bars = np.concatenate([[1.0], _alpha_bars[:-1]])
_LA = np.log(_alphas + _EPS).astype(np.float32)
_L1MA = np.log(1.0 - _alphas + _EPS).astype(np.float32)
_LPAB = np.log(_prev_alpha_bars + _EPS).astype(np.float32)
_L1MPAB = np.log(1.0 - _prev_alpha_bars + _EPS).astype(np.float32)
_C_INIT = np.float32(np.maximum(np.log(1.0 / _NCLASS + _EPS), -30.0))
_LOG_EPS30 = np.log(np.float32(_EPS))


_TABLE = np.zeros((64, 8), dtype=np.float32)
_TABLE[:_T, 0] = _LA
_TABLE[:_T, 1] = (_LOG_EPS30 + _LA).astype(np.float32)
_TABLE[:_T, 2] = (_L1MA + _C_INIT).astype(np.float32)
_TABLE[:_T, 3] = _LPAB
_TABLE[:_T, 4] = (_L1MPAB + _C_INIT).astype(np.float32)
_TABLE[0, 5] = 1.0


_NW = 32
_BPW = _N // _NW


@functools.cache
def _make_sc_gather_t():
    @functools.partial(
        pl.kernel,
        mesh=plsc.VectorSubcoreMesh(core_axis_name="c", subcore_axis_name="s"),
        out_type=jax.ShapeDtypeStruct((_N,), jnp.int32),
        scratch_types=[
            pltpu.VMEM((_B,), jnp.int32),
            pltpu.VMEM((_BPW,), jnp.int32),
            pltpu.VMEM((_BPW,), jnp.int32),
        ],
        compiler_params=pltpu.CompilerParams(needs_layout_passes=False),
    )
    def _sc_gather_t(ts_hbm, batch_hbm, out_hbm, ts_v, b_v, t_v):
        wid = lax.axis_index("s") * 2 + lax.axis_index("c")
        base = wid * _BPW
        pltpu.sync_copy(ts_hbm, ts_v)
        pltpu.sync_copy(batch_hbm.at[pl.ds(base, _BPW)], b_v)
        for g in range(_BPW // 16):
            idx = b_v[pl.ds(g * 16, 16)]
            t_v[pl.ds(g * 16, 16)] = plsc.load_gather(ts_v, [idx])
        pltpu.sync_copy(t_v, out_hbm.at[pl.ds(base, _BPW)])

    return _sc_gather_t


def _body(x0_ref, u_ref, xt_ref, t_ref, tab_ref, lp_ref, s_ref):
    r = x0_ref.shape[0]
    lane = jax.lax.broadcasted_iota(jnp.int32, (1, _NCLASS), 1)

    tab = tab_ref[...]
    la_c = tab[:, 0:1]
    lae_c = tab[:, 1:2]
    l1_c = tab[:, 2:3]
    m1 = jnp.maximum(la_c, l1_c)
    lxt_c = m1 + jnp.log(jnp.exp(la_c - m1) + jnp.exp(l1_c - m1))
    m2 = jnp.maximum(lae_c, l1_c)
    loth_c = m2 + jnp.log(jnp.exp(lae_c - m2) + jnp.exp(l1_c - m2))
    mat = jnp.concatenate([lxt_c, loth_c, tab[:, 3:6]], axis=1)
    lane64 = jax.lax.broadcasted_iota(jnp.int32, (1, 64), 1)

    x0 = x0_ref[...]
    oh_t = (t_ref[...] == lane64).astype(jnp.float32)
    coefs = jax.lax.dot_general(oh_t, mat, (((1,), (0,)), ((), ())),
                                precision=jax.lax.Precision.HIGHEST,
                                preferred_element_type=jnp.float32)
    lxt = coefs[:, 0:1]
    loth = coefs[:, 1:2]
    lpab = coefs[:, 2:3]
    r2 = coefs[:, 3:4]
    tzm = coefs[:, 4:5] > 0.5

    xtm = xt_ref[...] == lane
    left = jnp.where(xtm, lxt, loth)
    a = lpab + x0
    m = jnp.maximum(a, r2)
    right = m + jnp.log(jnp.exp(a - m) + jnp.exp(r2 - m))
    lp = left + right
    rowmax = jnp.max(lp, axis=1, keepdims=True)
    lse = rowmax + jnp.log(
        jnp.sum(jnp.exp(lp - rowmax), axis=1, keepdims=True))
    lp_out = jnp.where(tzm, x0, lp - lse)
    lp_ref[...] = lp_out

    g = -jnp.log(-jnp.log(u_ref[...] + jnp.float32(_EPS))
                 + jnp.float32(_EPS))
    sel = jnp.where(tzm, x0, g + lp_out)
    s_ref[...] = jnp.argmax(sel, axis=1).astype(jnp.int32)[:, None]


def kernel(x0_logprob, uniform_noise, xt, timestep, batch):
    r = 4096
    grid = (_N // r,)
    tvals = _make_sc_gather_t()(timestep, batch)
    xt2 = xt.reshape(_N, 1)
    t2 = tvals.reshape(_N, 1)
    tab = jnp.asarray(_TABLE)
    lp, s2 = pl.pallas_call(
        _body,
        grid=grid,
        in_specs=[
            pl.BlockSpec((r, _NCLASS), lambda i: (i, 0)),
            pl.BlockSpec((r, _NCLASS), lambda i: (i, 0)),
            pl.BlockSpec((r, 1), lambda i: (i, 0)),
            pl.BlockSpec((r, 1), lambda i: (i, 0)),
            pl.BlockSpec((64, 8), lambda i: (0, 0)),
        ],
        out_specs=[
            pl.BlockSpec((r, _NCLASS), lambda i: (i, 0)),
            pl.BlockSpec((r, 1), lambda i: (i, 0)),
        ],
        out_shape=[
            jax.ShapeDtypeStruct((_N, _NCLASS), jnp.float32),
            jax.ShapeDtypeStruct((_N, 1), jnp.int32),
        ],
        compiler_params=pltpu.CompilerParams(dimension_semantics=("parallel",)),
    )(x0_logprob, uniform_noise, xt2, t2, tab)
    return lp, s2.reshape(_N)


# speedup vs baseline: 1.7381x; 1.0173x over previous
import functools

import jax
import jax.numpy as jnp
import numpy as np
from jax import lax
from jax.experimental import pallas as pl
from jax.experimental.pallas import tpu as pltpu
from jax.experimental.pallas import tpu_sc as plsc

_N = 32768
_NCLASS = 128
_B = 128
_T = 50
_EPS = 1e-30

_betas = np.array([0.0004 * (k + 1) for k in range(_T)], dtype=np.float64)
_alphas = 1.0 - _betas
_alpha_bars = np.cumprod(_alphas, axis=0)
_prev_alpha_bars = np.concatenate([[1.0], _alpha_bars[:-1]])
_LA = np.log(_alphas + _EPS).astype(np.float32)
_L1MA = np.log(1.0 - _alphas + _EPS).astype(np.float32)
_LPAB = np.log(_prev_alpha_bars + _EPS).astype(np.float32)
_L1MPAB = np.log(1.0 - _prev_alpha_bars + _EPS).astype(np.float32)
_C_INIT = np.float32(np.maximum(np.log(1.0 / _NCLASS + _EPS), -30.0))
_LOG_EPS30 = np.log(np.float32(_EPS))


_TABLE = np.zeros((64, 8), dtype=np.float32)
_TABLE[:_T, 0] = _LA
_TABLE[:_T, 1] = (_LOG_EPS30 + _LA).astype(np.float32)
_TABLE[:_T, 2] = (_L1MA + _C_INIT).astype(np.float32)
_TABLE[:_T, 3] = _LPAB
_TABLE[:_T, 4] = (_L1MPAB + _C_INIT).astype(np.float32)
_TABLE[0, 5] = 1.0


_NW = 16
_BPW = _N // _NW


@functools.cache
def _make_sc_gather_t():
    @functools.partial(
        pl.kernel,
        mesh=plsc.VectorSubcoreMesh(core_axis_name="c", subcore_axis_name="s", num_cores=1),
        out_type=jax.ShapeDtypeStruct((_N,), jnp.int32),
        scratch_types=[
            pltpu.VMEM((_B,), jnp.int32),
            pltpu.VMEM((_BPW,), jnp.int32),
            pltpu.VMEM((_BPW,), jnp.int32),
        ],
        compiler_params=pltpu.CompilerParams(needs_layout_passes=False),
    )
    def _sc_gather_t(ts_hbm, batch_hbm, out_hbm, ts_v, b_v, t_v):
        wid = lax.axis_index("s") + lax.axis_index("c") * 0
        base = wid * _BPW
        pltpu.sync_copy(ts_hbm, ts_v)
        pltpu.sync_copy(batch_hbm.at[pl.ds(base, _BPW)], b_v)
        for g in range(_BPW // 16):
            idx = b_v[pl.ds(g * 16, 16)]
            t_v[pl.ds(g * 16, 16)] = plsc.load_gather(ts_v, [idx])
        pltpu.sync_copy(t_v, out_hbm.at[pl.ds(base, _BPW)])

    return _sc_gather_t


def _body(x0_ref, u_ref, xt_ref, t_ref, tab_ref, lp_ref, s_ref):
    r = x0_ref.shape[0]
    lane = jax.lax.broadcasted_iota(jnp.int32, (1, _NCLASS), 1)

    tab = tab_ref[...]
    la_c = tab[:, 0:1]
    lae_c = tab[:, 1:2]
    l1_c = tab[:, 2:3]
    m1 = jnp.maximum(la_c, l1_c)
    lxt_c = m1 + jnp.log(jnp.exp(la_c - m1) + jnp.exp(l1_c - m1))
    m2 = jnp.maximum(lae_c, l1_c)
    loth_c = m2 + jnp.log(jnp.exp(lae_c - m2) + jnp.exp(l1_c - m2))
    mat = jnp.concatenate([lxt_c, loth_c, tab[:, 3:6]], axis=1)
    lane64 = jax.lax.broadcasted_iota(jnp.int32, (1, 64), 1)

    x0 = x0_ref[...]
    oh_t = (t_ref[...] == lane64).astype(jnp.float32)
    coefs = jax.lax.dot_general(oh_t, mat, (((1,), (0,)), ((), ())),
                                precision=jax.lax.Precision.HIGHEST,
                                preferred_element_type=jnp.float32)
    lxt = coefs[:, 0:1]
    loth = coefs[:, 1:2]
    lpab = coefs[:, 2:3]
    r2 = coefs[:, 3:4]
    tzm = coefs[:, 4:5] > 0.5

    xtm = xt_ref[...] == lane
    left = jnp.where(xtm, lxt, loth)
    a = lpab + x0
    m = jnp.maximum(a, r2)
    right = m + jnp.log(jnp.exp(a - m) + jnp.exp(r2 - m))
    lp = left + right
    rowmax = jnp.max(lp, axis=1, keepdims=True)
    lse = rowmax + jnp.log(
        jnp.sum(jnp.exp(lp - rowmax), axis=1, keepdims=True))
    lp_out = jnp.where(tzm, x0, lp - lse)
    lp_ref[...] = lp_out

    g = -jnp.log(-jnp.log(u_ref[...] + jnp.float32(_EPS))
                 + jnp.float32(_EPS))
    sel = jnp.where(tzm, x0, g + lp_out)
    s_ref[...] = jnp.argmax(sel, axis=1).astype(jnp.int32)[:, None]


def kernel(x0_logprob, uniform_noise, xt, timestep, batch):
    r = 4096
    grid = (_N // r,)
    tvals = _make_sc_gather_t()(timestep, batch)
    xt2 = xt.reshape(_N, 1)
    t2 = tvals.reshape(_N, 1)
    tab = jnp.asarray(_TABLE)
    lp, s2 = pl.pallas_call(
        _body,
        grid=grid,
        in_specs=[
            pl.BlockSpec((r, _NCLASS), lambda i: (i, 0)),
            pl.BlockSpec((r, _NCLASS), lambda i: (i, 0)),
            pl.BlockSpec((r, 1), lambda i: (i, 0)),
            pl.BlockSpec((r, 1), lambda i: (i, 0)),
            pl.BlockSpec((64, 8), lambda i: (0, 0)),
        ],
        out_specs=[
            pl.BlockSpec((r, _NCLASS), lambda i: (i, 0)),
            pl.BlockSpec((r, 1), lambda i: (i, 0)),
        ],
        out_shape=[
            jax.ShapeDtypeStruct((_N, _NCLASS), jnp.float32),
            jax.ShapeDtypeStruct((_N, 1), jnp.int32),
        ],
        compiler_params=pltpu.CompilerParams(dimension_semantics=("parallel",)),
    )(x0_logprob, uniform_noise, xt2, t2, tab)
    return lp, s2.reshape(_N)
